# Initial kernel scaffold; baseline (speedup 1.0000x reference)
import jax
import jax.numpy as jnp
from jax.experimental import pallas as pl


def kernel(boxes, scores, reg):
    raise NotImplementedError("write your pallas kernel here")


# speedup vs baseline: 33.9801x; 33.9801x over previous
import jax
import jax.numpy as jnp
from jax.experimental import pallas as pl
from jax.experimental.pallas import tpu as pltpu

_N = 5000
_NP = 5120
_CHUNK = 128
_SCORE_T = 0.7
_NMS_T = 0.7


def _square_round(x1, y1, x2, y2):
    h = y2 - y1 + 1.0
    w = x2 - x1 + 1.0
    m = jnp.maximum(h, w)
    nx1 = x1 + w * 0.5 - m * 0.5
    ny1 = y1 + h * 0.5 - m * 0.5
    nx2 = nx1 + m - 1.0
    ny2 = ny1 + m - 1.0
    return (jnp.round(nx1), jnp.round(ny1), jnp.round(nx2), jnp.round(ny2))


def _nms_body(boxes_r, boxesT, prob_r, prob_c, regT, out_ref, feat, keepv):
    x1r, y1r, x2r, y2r = (boxes_r[:, k : k + 1] for k in range(4))
    sx1r, sy1r, sx2r, sy2r = _square_round(x1r, y1r, x2r, y2r)
    arear = (sx2r - sx1r + 1.0) * (sy2r - sy1r + 1.0)
    effr = jnp.where(prob_r[:] > _SCORE_T, prob_r[:], -1.0)
    feat[:, 0:1] = sx1r
    feat[:, 1:2] = sy1r
    feat[:, 2:3] = sx2r
    feat[:, 3:4] = sy2r
    feat[:, 4:5] = arear
    feat[:, 5:6] = effr

    x1c, y1c, x2c, y2c = (boxesT[k : k + 1, :] for k in range(4))
    sx1c, sy1c, sx2c, sy2c = _square_round(x1c, y1c, x2c, y2c)
    areac = (sx2c - sx1c + 1.0) * (sy2c - sy1c + 1.0)
    probc = prob_c[:]
    candc = probc > _SCORE_T
    effc = jnp.where(candc, probc, -1.0)
    idxc = jax.lax.broadcasted_iota(jnp.int32, (1, _NP), 1)

    nchunks = _NP // _CHUNK

    def count_conflicts(keep):
        def chunk(c, acc):
            base = c * _CHUNK
            f = feat[pl.ds(base, _CHUNK), :]
            cx1, cy1 = f[:, 0:1], f[:, 1:2]
            cx2, cy2 = f[:, 2:3], f[:, 3:4]
            car, ceff = f[:, 4:5], f[:, 5:6]
            xx1 = jnp.maximum(cx1, sx1c)
            yy1 = jnp.maximum(cy1, sy1c)
            xx2 = jnp.minimum(cx2, sx2c)
            yy2 = jnp.minimum(cy2, sy2c)
            w = jnp.maximum(0.0, xx2 - xx1 + 1.0)
            h = jnp.maximum(0.0, yy2 - yy1 + 1.0)
            inter = w * h
            iou = inter / (car + areac - inter + 1e-10)
            idxr = jax.lax.broadcasted_iota(jnp.int32, (_CHUNK, 1), 0) + base
            prec = (ceff > effc) | ((ceff == effc) & (idxr < idxc))
            conf = jnp.where((iou > _NMS_T) & prec, 1.0, 0.0)
            krow = keepv[:, pl.ds(base, _CHUNK)]
            return acc + jnp.dot(krow, conf, preferred_element_type=jnp.float32)

        return jax.lax.fori_loop(0, nchunks, chunk, jnp.zeros((1, _NP), jnp.float32))

    keep0 = jnp.where(candc, 1.0, 0.0)

    def cond(carry):
        return carry[1]

    def body(carry):
        keep, _ = carry
        keepv[:, :] = keep
        nkeep = jnp.where(candc & (count_conflicts(keep) == 0.0), 1.0, 0.0)
        return nkeep, jnp.any(nkeep != keep)

    keep, _ = jax.lax.while_loop(cond, body, (keep0, jnp.bool_(True)))

    bbw = sx2c - sx1c + 1.0
    bbh = sy2c - sy1c + 1.0
    out_ref[0:1, :] = (sx1c + regT[0:1, :] * bbw) * keep
    out_ref[1:2, :] = (sy1c + regT[1:2, :] * bbh) * keep
    out_ref[2:3, :] = (sx2c + regT[2:3, :] * bbw) * keep
    out_ref[3:4, :] = (sy2c + regT[3:4, :] * bbh) * keep
    out_ref[4:5, :] = probc * keep
    out_ref[5:8, :] = jnp.zeros((3, _NP), jnp.float32)


def kernel(boxes, scores, reg):
    pad = _NP - _N
    boxes_p = jnp.pad(boxes, ((0, pad), (0, 0)))
    prob = jnp.pad(scores[:, 1:2], ((0, pad), (0, 0)))
    regT = jnp.pad(reg, ((0, pad), (0, 0))).T
    outT = pl.pallas_call(
        _nms_body,
        out_shape=jax.ShapeDtypeStruct((8, _NP), jnp.float32),
        scratch_shapes=[
            pltpu.VMEM((_NP, 8), jnp.float32),
            pltpu.VMEM((1, _NP), jnp.float32),
        ],
    )(boxes_p, boxes_p.T, prob, prob.T, regT)
    return outT[:5, :_N].T
